# baseline (speedup 1.0000x reference)
import functools

import jax
import jax.numpy as jnp
from jax import lax
from jax.experimental import pallas as pl
from jax.experimental.pallas import tpu as pltpu
from jax.experimental.pallas import tpu_sc as plsc

F32 = jnp.float32

_N = 10000
_E = 320000
_D = 128
_HEADS = 16
_HEAD_DIM = 8
_NW = 32
_EPW = _E // _NW
_CH = 80
_NCHUNK = _EPW // _CH
_NA = 10240
_RPS = _NA // 16
_RCH = 128
_BN = 400
_BE = 512


def _node_body(h_ref, w1d_ref, w1s_ref, w1q_ref, b1q_ref, gq_ref, beq_ref,
               w2q_ref, b2q_ref, td_ref, ts_ref):
    hb = h_ref[...]
    ad = jnp.dot(hb, w1d_ref[...], preferred_element_type=F32)
    ts_ref[...] = jnp.dot(hb, w1s_ref[...], preferred_element_type=F32)
    hq = jnp.dot(hb, w1q_ref[...], preferred_element_type=F32) + b1q_ref[...]
    mu = jnp.mean(hq, axis=-1, keepdims=True)
    var = jnp.mean((hq - mu) ** 2, axis=-1, keepdims=True)
    hqn = (hq - mu) * lax.rsqrt(var + 1e-5) * gq_ref[...] + beq_ref[...]
    q = jnp.dot(jnp.maximum(hqn, 0.0), w2q_ref[...],
                preferred_element_type=F32) + b2q_ref[...]
    td_ref[...] = jnp.concatenate([ad, q], axis=1)


def _node_pass(h, w1d, w1s, w1q, b1q, gq, beq, w2q, b2q):
    nb = _N // _BN
    full = lambda shape: pl.BlockSpec(shape, lambda i: (0, 0))
    return pl.pallas_call(
        _node_body,
        grid=(nb,),
        in_specs=[
            pl.BlockSpec((_BN, _D), lambda i: (i, 0)),
            full((_D, 256)), full((_D, 256)), full((_D, _D)),
            full((1, _D)), full((1, _D)), full((1, _D)),
            full((_D, _D)), full((1, _D)),
        ],
        out_specs=[
            pl.BlockSpec((_BN, 384), lambda i: (i, 0)),
            pl.BlockSpec((_BN, 256), lambda i: (i, 0)),
        ],
        out_shape=[
            jax.ShapeDtypeStruct((_N, 384), F32),
            jax.ShapeDtypeStruct((_N, 256), F32),
        ],
    )(h, w1d, w1s, w1q, b1q, gq, beq, w2q, b2q)


def _sc_gather(td, ts, dst, src):
    mesh = plsc.VectorSubcoreMesh(core_axis_name="c", subcore_axis_name="s")

    @functools.partial(
        pl.kernel, mesh=mesh,
        out_type=(jax.ShapeDtypeStruct((_E, 384), F32),
                  jax.ShapeDtypeStruct((_E, 256), F32)),
        scratch_types=[
            pltpu.VMEM((_CH,), jnp.int32),
            pltpu.VMEM((_CH,), jnp.int32),
            pltpu.VMEM((_CH, 384), F32),
            pltpu.VMEM((_CH, 256), F32),
            pltpu.SemaphoreType.DMA,
            pltpu.SemaphoreType.DMA,
        ],
    )
    def gather_k(td_hbm, ts_hbm, dst_hbm, src_hbm, gd_hbm, gs_hbm,
                 idxd, idxs, gd_v, gs_v, sem1, sem2):
        wid = lax.axis_index("s") * 2 + lax.axis_index("c")
        base0 = wid * _EPW

        def body(i, carry):
            base = base0 + i * _CH
            pltpu.sync_copy(dst_hbm.at[pl.ds(base, _CH)], idxd)
            pltpu.sync_copy(src_hbm.at[pl.ds(base, _CH)], idxs)
            cp1 = pltpu.async_copy(td_hbm.at[idxd], gd_v, sem1)
            cp2 = pltpu.async_copy(ts_hbm.at[idxs], gs_v, sem2)
            cp1.wait()
            cp2.wait()
            pltpu.sync_copy(gd_v, gd_hbm.at[pl.ds(base, _CH)])
            pltpu.sync_copy(gs_v, gs_hbm.at[pl.ds(base, _CH)])
            return carry

        lax.fori_loop(0, _NCHUNK, body, 0)

    return gather_k(td, ts, dst, src)


def _edge_body(gd_ref, gs_ref, rf_ref, ef_ref, rx_ref,
               w1ef_ref, w1rf_ref, b1_ref, g_ref, be_ref,
               w2k_ref, b2k_ref, w2v_ref, b2v_ref, eww_ref, ewb_ref, ssel_ref,
               vsr_ref, lmax_ref):
    gd = gd_ref[...]
    rf = rf_ref[...]
    pre = (gd[:, :256] + gs_ref[...]
           + jnp.dot(ef_ref[...], w1ef_ref[...], preferred_element_type=F32)
           + jnp.dot(rf, w1rf_ref[...], preferred_element_type=F32)
           + b1_ref[...])

    def ln_relu(x, g, be):
        mu = jnp.mean(x, axis=-1, keepdims=True)
        var = jnp.mean((x - mu) ** 2, axis=-1, keepdims=True)
        return jnp.maximum((x - mu) * lax.rsqrt(var + 1e-5) * g + be, 0.0)

    rk = ln_relu(pre[:, :_D], g_ref[...][:, :_D], be_ref[...][:, :_D])
    rv = ln_relu(pre[:, _D:], g_ref[...][:, _D:], be_ref[...][:, _D:])
    kvec = jnp.dot(rk, w2k_ref[...], preferred_element_type=F32) + b2k_ref[...]
    v16 = jnp.dot(rv, w2v_ref[...], preferred_element_type=F32) + b2v_ref[...]
    ew = jax.nn.sigmoid(jnp.dot(rf, eww_ref[...], preferred_element_type=F32)
                        + ewb_ref[...])
    vs = v16 * ew
    prod = gd[:, 256:384] * kvec * (1.0 / jnp.sqrt(jnp.float32(_HEAD_DIM)))
    logits = jnp.dot(prod, ssel_ref[...], preferred_element_type=F32)
    rx = rx_ref[...]
    vsr_ref[...] = jnp.concatenate(
        [logits, vs * rx[:, 0:1], vs * rx[:, 1:2], vs * rx[:, 2:3],
         jnp.zeros((_BE, 64), F32)], axis=1)
    bm = jnp.max(logits, axis=0, keepdims=True)

    @pl.when(pl.program_id(0) == 0)
    def _():
        lmax_ref[...] = bm

    @pl.when(pl.program_id(0) != 0)
    def _():
        lmax_ref[...] = jnp.maximum(lmax_ref[...], bm)


def _edge_pass(gd, gs, rf, ef, rx, w1ef, w1rf, b1, g, be,
               w2k, b2k, w2v, b2v, eww, ewb, ssel):
    nb = _E // _BE
    full = lambda shape: pl.BlockSpec(shape, lambda i: (0, 0))
    return pl.pallas_call(
        _edge_body,
        grid=(nb,),
        in_specs=[
            pl.BlockSpec((_BE, 384), lambda i: (i, 0)),
            pl.BlockSpec((_BE, 256), lambda i: (i, 0)),
            pl.BlockSpec((_BE, 16), lambda i: (i, 0)),
            pl.BlockSpec((_BE, 4), lambda i: (i, 0)),
            pl.BlockSpec((_BE, 3), lambda i: (i, 0)),
            full((4, 256)), full((16, 256)), full((1, 256)),
            full((1, 256)), full((1, 256)),
            full((_D, _D)), full((1, _D)), full((_D, 16)), full((1, 16)),
            full((16, 1)), full((1, 1)), full((_D, 16)),
        ],
        out_specs=[
            pl.BlockSpec((_BE, 128), lambda i: (i, 0)),
            pl.BlockSpec((1, 16), lambda i: (0, 0)),
        ],
        out_shape=[
            jax.ShapeDtypeStruct((_E, 128), F32),
            jax.ShapeDtypeStruct((1, 16), F32),
        ],
    )(gd, gs, rf, ef, rx, w1ef, w1rf, b1, g, be,
      w2k, b2k, w2v, b2v, eww, ewb, ssel)


def _sc_scatter(vsr, dst, lmax):
    mesh = plsc.VectorSubcoreMesh(core_axis_name="c", subcore_axis_name="s")

    @functools.partial(
        pl.kernel, mesh=mesh,
        out_type=jax.ShapeDtypeStruct((2, _NA, 128), F32),
        scratch_types=[
            pltpu.VMEM_SHARED((_NA, 128), F32),
            pltpu.VMEM((_CH, 128), F32),
            pltpu.VMEM((_CH,), jnp.int32),
            pltpu.VMEM((_RCH, 128), F32),
            pltpu.VMEM((1, 16), F32),
        ],
    )
    def scatter_k(vsr_hbm, dst_hbm, lmax_hbm, out_hbm,
                  acc, vbuf, ibuf, zbuf, gbuf):
        c = lax.axis_index("c")
        s = lax.axis_index("s")
        wid = s * 2 + c

        def zb(j, carry):
            for t in range(8):
                zbuf[j, 16 * t:16 * (t + 1)] = jnp.zeros((16,), F32)
            return carry

        lax.fori_loop(0, _RCH, zb, 0)

        def zc(j, carry):
            pltpu.sync_copy(zbuf, acc.at[pl.ds(s * _RPS + j * _RCH, _RCH)])
            return carry

        lax.fori_loop(0, _RPS // _RCH, zc, 0)
        pltpu.sync_copy(lmax_hbm, gbuf)
        plsc.subcore_barrier()

        gvec = gbuf[0, :]
        base0 = wid * _EPW

        def body(i, carry):
            base = base0 + i * _CH
            pltpu.sync_copy(vsr_hbm.at[pl.ds(base, _CH)], vbuf)
            pltpu.sync_copy(dst_hbm.at[pl.ds(base, _CH)], ibuf)

            def ed(j, carry2):
                e = jnp.exp(vbuf[j, 0:16] - gvec)
                vbuf[j, 0:16] = e
                vbuf[j, 16:32] = e * vbuf[j, 16:32]
                vbuf[j, 32:48] = e * vbuf[j, 32:48]
                vbuf[j, 48:64] = e * vbuf[j, 48:64]
                return carry2

            lax.fori_loop(0, _CH, ed, 0)
            pltpu.sync_copy(vbuf, acc.at[ibuf], add=True)
            return carry

        lax.fori_loop(0, _NCHUNK, body, 0)
        plsc.subcore_barrier()

        def dr(j, carry):
            rs = s * _RPS + j * _RCH
            pltpu.sync_copy(acc.at[pl.ds(rs, _RCH)], zbuf)
            pltpu.sync_copy(zbuf, out_hbm.at[c, pl.ds(rs, _RCH)])
            return carry

        lax.fori_loop(0, _RPS // _RCH, dr, 0)

    return scatter_k(vsr, dst, lmax)


def _comb_body(p0_ref, p1_ref, ssum_ref, o_ref):
    a = (p0_ref[...] + p1_ref[...]).reshape(_BN, 128)
    den = a[:, 0:16] + 1e-16
    ratio = a[:, 16:64] / jnp.concatenate([den, den, den], axis=1)
    o_ref[...] = jnp.dot(ratio, ssum_ref[...],
                         preferred_element_type=F32) * (1.0 / _HEADS)


def _combine(partials, ssum):
    nb = _N // _BN
    return pl.pallas_call(
        _comb_body,
        grid=(nb,),
        in_specs=[
            pl.BlockSpec((1, _BN, 128), lambda i: (0, i, 0)),
            pl.BlockSpec((1, _BN, 128), lambda i: (1, i, 0)),
            pl.BlockSpec((48, 3), lambda i: (0, 0)),
        ],
        out_specs=pl.BlockSpec((_BN, 3), lambda i: (i, 0)),
        out_shape=jax.ShapeDtypeStruct((_N, 3), F32),
    )(partials, partials, ssum)


def kernel(h, rel_x, r_feat, edge_feat, edge_index,
           xk_W1, xk_b1, xk_g, xk_be, xk_W2, xk_b2,
           xv_W1, xv_b1, xv_g, xv_be, xv_W2, xv_b2,
           xq_W1, xq_b1, xq_g, xq_be, xq_W2, xq_b2,
           ew_W, ew_b):
    src = edge_index[0]
    dst = edge_index[1]

    w1d = jnp.concatenate([xk_W1[20:148], xv_W1[20:148]], axis=1)
    w1s = jnp.concatenate([xk_W1[148:276], xv_W1[148:276]], axis=1)
    w1ef = jnp.concatenate([xk_W1[0:4], xv_W1[0:4]], axis=1)
    w1rf = jnp.concatenate([xk_W1[4:20], xv_W1[4:20]], axis=1)
    b1 = jnp.concatenate([xk_b1, xv_b1])[None, :]
    g = jnp.concatenate([xk_g, xv_g])[None, :]
    be = jnp.concatenate([xk_be, xv_be])[None, :]
    lanes = jnp.arange(_D)
    ssel = (lanes[:, None] // _HEAD_DIM ==
            jnp.arange(_HEADS)[None, :]).astype(F32)
    lanes48 = jnp.arange(48)
    ssum = (lanes48[:, None] // _HEADS == jnp.arange(3)[None, :]).astype(F32)

    td, ts = _node_pass(h, w1d, w1s, xq_W1, xq_b1[None, :], xq_g[None, :],
                        xq_be[None, :], xq_W2, xq_b2[None, :])
    gd, gs = _sc_gather(td, ts, dst, src)
    vsr, lmax = _edge_pass(gd, gs, r_feat, edge_feat, rel_x,
                           w1ef, w1rf, b1, g, be,
                           xk_W2, xk_b2[None, :], xv_W2, xv_b2[None, :],
                           ew_W, ew_b[None, :], ssel)
    partials = _sc_scatter(vsr, dst, lmax)
    return _combine(partials, ssum)


# speedup vs baseline: 1.1414x; 1.1414x over previous
import functools

import jax
import jax.numpy as jnp
from jax import lax
from jax.experimental import pallas as pl
from jax.experimental.pallas import tpu as pltpu
from jax.experimental.pallas import tpu_sc as plsc

F32 = jnp.float32

_N = 10000
_E = 320000
_D = 128
_HEADS = 16
_HEAD_DIM = 8
_NW = 32
_EPW = _E // _NW
_CH = 80
_NCHUNK = _EPW // _CH
_NA = 10240
_RPS = _NA // 16
_RCH = 128
_BN = 400
_BE = 512


def _node_body(h_ref, w1d_ref, w1s_ref, w1q_ref, b1q_ref, gq_ref, beq_ref,
               w2q_ref, b2q_ref, td_ref, ts_ref):
    hb = h_ref[...]
    ad = jnp.dot(hb, w1d_ref[...], preferred_element_type=F32)
    ts_ref[...] = jnp.dot(hb, w1s_ref[...], preferred_element_type=F32)
    hq = jnp.dot(hb, w1q_ref[...], preferred_element_type=F32) + b1q_ref[...]
    mu = jnp.mean(hq, axis=-1, keepdims=True)
    var = jnp.mean((hq - mu) ** 2, axis=-1, keepdims=True)
    hqn = (hq - mu) * lax.rsqrt(var + 1e-5) * gq_ref[...] + beq_ref[...]
    q = jnp.dot(jnp.maximum(hqn, 0.0), w2q_ref[...],
                preferred_element_type=F32) + b2q_ref[...]
    td_ref[...] = jnp.concatenate([ad, q], axis=1)


def _node_pass(h, w1d, w1s, w1q, b1q, gq, beq, w2q, b2q):
    nb = _N // _BN
    full = lambda shape: pl.BlockSpec(shape, lambda i: (0, 0))
    return pl.pallas_call(
        _node_body,
        grid=(nb,),
        in_specs=[
            pl.BlockSpec((_BN, _D), lambda i: (i, 0)),
            full((_D, 256)), full((_D, 256)), full((_D, _D)),
            full((1, _D)), full((1, _D)), full((1, _D)),
            full((_D, _D)), full((1, _D)),
        ],
        out_specs=[
            pl.BlockSpec((_BN, 384), lambda i: (i, 0)),
            pl.BlockSpec((_BN, 256), lambda i: (i, 0)),
        ],
        out_shape=[
            jax.ShapeDtypeStruct((_N, 384), F32),
            jax.ShapeDtypeStruct((_N, 256), F32),
        ],
    )(h, w1d, w1s, w1q, b1q, gq, beq, w2q, b2q)


def _sc_gather(td, ts, dst, src):
    mesh = plsc.VectorSubcoreMesh(core_axis_name="c", subcore_axis_name="s")

    @functools.partial(
        pl.kernel, mesh=mesh,
        out_type=(jax.ShapeDtypeStruct((_E, 256), F32),
                  jax.ShapeDtypeStruct((_E, 128), F32)),
        scratch_types=[
            pltpu.VMEM((_CH,), jnp.int32),
            pltpu.VMEM((_CH,), jnp.int32),
            pltpu.VMEM((_CH, 256), F32),
            pltpu.VMEM((_CH, 128), F32),
            pltpu.SemaphoreType.DMA,
            pltpu.SemaphoreType.DMA,
        ],
    )
    def gather_k(td_hbm, ts_hbm, dst_hbm, src_hbm, gd_hbm, gs_hbm,
                 idxd, idxs, gd_v, gs_v, sem1, sem2):
        wid = lax.axis_index("s") * 2 + lax.axis_index("c")
        base0 = wid * _EPW

        def body(i, carry):
            base = base0 + i * _CH
            pltpu.sync_copy(dst_hbm.at[pl.ds(base, _CH)], idxd)
            pltpu.sync_copy(src_hbm.at[pl.ds(base, _CH)], idxs)
            cp1 = pltpu.async_copy(td_hbm.at[idxd], gd_v, sem1)
            cp2 = pltpu.async_copy(ts_hbm.at[idxs], gs_v, sem2)
            cp1.wait()
            cp2.wait()
            pltpu.sync_copy(gd_v, gd_hbm.at[pl.ds(base, _CH)])
            pltpu.sync_copy(gs_v, gs_hbm.at[pl.ds(base, _CH)])
            return carry

        lax.fori_loop(0, _NCHUNK, body, 0)

    return gather_k(td, ts, dst, src)


def _unpack_bf16_pair(packed):
    u = lax.bitcast_convert_type(packed, jnp.uint32)
    lo = lax.bitcast_convert_type(u << 16, F32)
    hi = lax.bitcast_convert_type(u & jnp.uint32(0xFFFF0000), F32)
    return jnp.concatenate([lo, hi], axis=1)


def _edge_body(gd_ref, gs_ref, rf_ref, ef_ref, rx_ref,
               w1ef_ref, w1rf_ref, b1_ref, g_ref, be_ref,
               w2k_ref, b2k_ref, w2v_ref, b2v_ref, eww_ref, ewb_ref, ssel_ref,
               vsr_ref, lmax_ref):
    gd = gd_ref[...]
    ad_av = _unpack_bf16_pair(gd[:, :128])
    gs = _unpack_bf16_pair(gs_ref[...])
    rf = rf_ref[...]
    pre = (ad_av + gs
           + jnp.dot(ef_ref[...], w1ef_ref[...], preferred_element_type=F32)
           + jnp.dot(rf, w1rf_ref[...], preferred_element_type=F32)
           + b1_ref[...])

    def ln_relu(x, g, be):
        mu = jnp.mean(x, axis=-1, keepdims=True)
        var = jnp.mean((x - mu) ** 2, axis=-1, keepdims=True)
        return jnp.maximum((x - mu) * lax.rsqrt(var + 1e-5) * g + be, 0.0)

    rk = ln_relu(pre[:, :_D], g_ref[...][:, :_D], be_ref[...][:, :_D])
    rv = ln_relu(pre[:, _D:], g_ref[...][:, _D:], be_ref[...][:, _D:])
    kvec = jnp.dot(rk, w2k_ref[...], preferred_element_type=F32) + b2k_ref[...]
    v16 = jnp.dot(rv, w2v_ref[...], preferred_element_type=F32) + b2v_ref[...]
    ew = jax.nn.sigmoid(jnp.dot(rf, eww_ref[...], preferred_element_type=F32)
                        + ewb_ref[...])
    vs = v16 * ew
    prod = gd[:, 128:256] * kvec * (1.0 / jnp.sqrt(jnp.float32(_HEAD_DIM)))
    logits = jnp.dot(prod, ssel_ref[...], preferred_element_type=F32)
    rx = rx_ref[...]
    vsr_ref[...] = jnp.concatenate(
        [logits, vs * rx[:, 0:1], vs * rx[:, 1:2], vs * rx[:, 2:3],
         jnp.zeros((_BE, 64), F32)], axis=1)
    bm = jnp.max(logits, axis=0, keepdims=True)

    @pl.when(pl.program_id(0) == 0)
    def _():
        lmax_ref[...] = bm

    @pl.when(pl.program_id(0) != 0)
    def _():
        lmax_ref[...] = jnp.maximum(lmax_ref[...], bm)


def _edge_pass(gd, gs, rf, ef, rx, w1ef, w1rf, b1, g, be,
               w2k, b2k, w2v, b2v, eww, ewb, ssel):
    nb = _E // _BE
    full = lambda shape: pl.BlockSpec(shape, lambda i: (0, 0))
    return pl.pallas_call(
        _edge_body,
        grid=(nb,),
        in_specs=[
            pl.BlockSpec((_BE, 256), lambda i: (i, 0)),
            pl.BlockSpec((_BE, 128), lambda i: (i, 0)),
            pl.BlockSpec((_BE, 16), lambda i: (i, 0)),
            pl.BlockSpec((_BE, 4), lambda i: (i, 0)),
            pl.BlockSpec((_BE, 3), lambda i: (i, 0)),
            full((4, 256)), full((16, 256)), full((1, 256)),
            full((1, 256)), full((1, 256)),
            full((_D, _D)), full((1, _D)), full((_D, 16)), full((1, 16)),
            full((16, 1)), full((1, 1)), full((_D, 16)),
        ],
        out_specs=[
            pl.BlockSpec((_BE, 128), lambda i: (i, 0)),
            pl.BlockSpec((1, 16), lambda i: (0, 0)),
        ],
        out_shape=[
            jax.ShapeDtypeStruct((_E, 128), F32),
            jax.ShapeDtypeStruct((1, 16), F32),
        ],
    )(gd, gs, rf, ef, rx, w1ef, w1rf, b1, g, be,
      w2k, b2k, w2v, b2v, eww, ewb, ssel)


def _sc_scatter(vsr, dst, lmax):
    mesh = plsc.VectorSubcoreMesh(core_axis_name="c", subcore_axis_name="s")

    @functools.partial(
        pl.kernel, mesh=mesh,
        out_type=jax.ShapeDtypeStruct((2, _NA, 128), F32),
        scratch_types=[
            pltpu.VMEM_SHARED((_NA, 128), F32),
            pltpu.VMEM((_CH, 128), F32),
            pltpu.VMEM((_CH,), jnp.int32),
            pltpu.VMEM((_RCH, 128), F32),
            pltpu.VMEM((1, 16), F32),
        ],
    )
    def scatter_k(vsr_hbm, dst_hbm, lmax_hbm, out_hbm,
                  acc, vbuf, ibuf, zbuf, gbuf):
        c = lax.axis_index("c")
        s = lax.axis_index("s")
        wid = s * 2 + c

        def zb(j, carry):
            for t in range(8):
                zbuf[j, 16 * t:16 * (t + 1)] = jnp.zeros((16,), F32)
            return carry

        lax.fori_loop(0, _RCH, zb, 0)

        def zc(j, carry):
            pltpu.sync_copy(zbuf, acc.at[pl.ds(s * _RPS + j * _RCH, _RCH)])
            return carry

        lax.fori_loop(0, _RPS // _RCH, zc, 0)
        pltpu.sync_copy(lmax_hbm, gbuf)
        plsc.subcore_barrier()

        gvec = gbuf[0, :]
        base0 = wid * _EPW

        def body(i, carry):
            base = base0 + i * _CH
            pltpu.sync_copy(vsr_hbm.at[pl.ds(base, _CH)], vbuf)
            pltpu.sync_copy(dst_hbm.at[pl.ds(base, _CH)], ibuf)

            def ed(j, carry2):
                e = jnp.exp(vbuf[j, 0:16] - gvec)
                vbuf[j, 0:16] = e
                vbuf[j, 16:32] = e * vbuf[j, 16:32]
                vbuf[j, 32:48] = e * vbuf[j, 32:48]
                vbuf[j, 48:64] = e * vbuf[j, 48:64]
                return carry2

            lax.fori_loop(0, _CH, ed, 0)
            pltpu.sync_copy(vbuf, acc.at[ibuf], add=True)
            return carry

        lax.fori_loop(0, _NCHUNK, body, 0)
        plsc.subcore_barrier()

        def dr(j, carry):
            rs = s * _RPS + j * _RCH
            pltpu.sync_copy(acc.at[pl.ds(rs, _RCH)], zbuf)
            pltpu.sync_copy(zbuf, out_hbm.at[c, pl.ds(rs, _RCH)])
            return carry

        lax.fori_loop(0, _RPS // _RCH, dr, 0)

    return scatter_k(vsr, dst, lmax)


def _comb_body(p0_ref, p1_ref, ssum_ref, o_ref):
    a = (p0_ref[...] + p1_ref[...]).reshape(_BN, 128)
    den = a[:, 0:16] + 1e-16
    ratio = a[:, 16:64] / jnp.concatenate([den, den, den], axis=1)
    o_ref[...] = jnp.dot(ratio, ssum_ref[...],
                         preferred_element_type=F32) * (1.0 / _HEADS)


def _combine(partials, ssum):
    nb = _N // _BN
    return pl.pallas_call(
        _comb_body,
        grid=(nb,),
        in_specs=[
            pl.BlockSpec((1, _BN, 128), lambda i: (0, i, 0)),
            pl.BlockSpec((1, _BN, 128), lambda i: (1, i, 0)),
            pl.BlockSpec((48, 3), lambda i: (0, 0)),
        ],
        out_specs=pl.BlockSpec((_BN, 3), lambda i: (i, 0)),
        out_shape=jax.ShapeDtypeStruct((_N, 3), F32),
    )(partials, partials, ssum)


def kernel(h, rel_x, r_feat, edge_feat, edge_index,
           xk_W1, xk_b1, xk_g, xk_be, xk_W2, xk_b2,
           xv_W1, xv_b1, xv_g, xv_be, xv_W2, xv_b2,
           xq_W1, xq_b1, xq_g, xq_be, xq_W2, xq_b2,
           ew_W, ew_b):
    src = edge_index[0]
    dst = edge_index[1]

    w1d = jnp.concatenate([xk_W1[20:148], xv_W1[20:148]], axis=1)
    w1s = jnp.concatenate([xk_W1[148:276], xv_W1[148:276]], axis=1)
    w1ef = jnp.concatenate([xk_W1[0:4], xv_W1[0:4]], axis=1)
    w1rf = jnp.concatenate([xk_W1[4:20], xv_W1[4:20]], axis=1)
    b1 = jnp.concatenate([xk_b1, xv_b1])[None, :]
    g = jnp.concatenate([xk_g, xv_g])[None, :]
    be = jnp.concatenate([xk_be, xv_be])[None, :]
    lanes = jnp.arange(_D)
    ssel = (lanes[:, None] // _HEAD_DIM ==
            jnp.arange(_HEADS)[None, :]).astype(F32)
    lanes48 = jnp.arange(48)
    ssum = (lanes48[:, None] // _HEADS == jnp.arange(3)[None, :]).astype(F32)

    td, ts = _node_pass(h, w1d, w1s, xq_W1, xq_b1[None, :], xq_g[None, :],
                        xq_be[None, :], xq_W2, xq_b2[None, :])

    def pack_pairs(t):
        w = t.shape[1] // 2
        lo = lax.bitcast_convert_type(
            t[:, :w].astype(jnp.bfloat16), jnp.uint16).astype(jnp.uint32)
        hi = lax.bitcast_convert_type(
            t[:, w:].astype(jnp.bfloat16), jnp.uint16).astype(jnp.uint32)
        return lax.bitcast_convert_type((hi << 16) | lo, F32)

    tdp = jnp.concatenate([pack_pairs(td[:, :256]), td[:, 256:]], axis=1)
    tsp = pack_pairs(ts)
    gd, gs = _sc_gather(tdp, tsp, dst, src)
    vsr, lmax = _edge_pass(gd, gs, r_feat, edge_feat, rel_x,
                           w1ef, w1rf, b1, g, be,
                           xk_W2, xk_b2[None, :], xv_W2, xv_b2[None, :],
                           ew_W, ew_b[None, :], ssel)
    partials = _sc_scatter(vsr, dst, lmax)
    return _combine(partials, ssum)
